# baseline (speedup 1.0000x reference)
import dataclasses
import functools

import jax
import jax.numpy as jnp
from jax import lax
from jax.experimental import pallas as pl
from jax.experimental.pallas import tpu as pltpu
from jax.experimental.pallas import tpu_sc as plsc

_B = 16384
_D = 128
_L = 16
_MARGIN = 1.0
_NC, _NS = 2, 16
_NW = _NC * _NS
_PER_W = _B // _NW
_C = 32
_NCHUNK = _PER_W // _C


def _nr_sqrt(u):
    bits = lax.bitcast_convert_type(u, jnp.int32)
    seed = jnp.int32(0x5F3759DF) - lax.shift_right_logical(bits, 1)
    y = lax.bitcast_convert_type(seed, jnp.float32)
    half_u = 0.5 * u
    for _ in range(3):
        y = y * (1.5 - half_u * y * y)
    return u * y


def _sc_hinge_partials(table, pos_idx, neg_idx):
    mesh = plsc.VectorSubcoreMesh(core_axis_name="c", subcore_axis_name="s")
    cp = pltpu.CompilerParams()
    fields = pltpu.CompilerParams.__dataclass_fields__
    if "needs_layout_passes" in fields:
        cp = dataclasses.replace(cp, needs_layout_passes=False)
    if "use_tc_tiling_on_sc" in fields:
        cp = dataclasses.replace(cp, use_tc_tiling_on_sc=False)

    @functools.partial(
        pl.kernel,
        compiler_params=cp,
        out_type=jax.ShapeDtypeStruct((_NW * _L,), jnp.float32),
        mesh=mesh,
        scratch_types=[
            pltpu.VMEM((_PER_W, 3), jnp.int32),
            pltpu.VMEM((_PER_W,), jnp.int32),
            pltpu.VMEM((_PER_W,), jnp.int32),
            pltpu.VMEM((_PER_W,), jnp.int32),
            pltpu.VMEM((_PER_W,), jnp.int32),
            pltpu.VMEM((_PER_W,), jnp.int32),
            pltpu.VMEM((_PER_W,), jnp.int32),
            pltpu.VMEM((2, _C, _D), jnp.float32),
            pltpu.VMEM((2, _C, _D), jnp.float32),
            pltpu.VMEM((2, _C, _D), jnp.float32),
            pltpu.VMEM((2, _C, _D), jnp.float32),
            pltpu.VMEM((2, _C, _D), jnp.float32),
            pltpu.VMEM((2, _C, _D), jnp.float32),
            pltpu.VMEM((_C, _L), jnp.float32),
            pltpu.VMEM((_C, _L), jnp.float32),
            pltpu.VMEM((_L,), jnp.float32),
            pltpu.SemaphoreType.DMA,
            pltpu.SemaphoreType.DMA,
        ],
    )
    def k(table_hbm, pos_hbm, neg_hbm, out_hbm,
          pix_v, c0_v, c1_v, c2_v, c3_v, c4_v, c5_v,
          hp_v, tp_v, rp_v, hn_v, tn_v, rn_v,
          pp_v, nn_v, hacc_v, g0, g1):
        gsem = (g0, g1)
        row_bufs = (hp_v, tp_v, rp_v, hn_v, tn_v, rn_v)
        col_bufs = (c0_v, c1_v, c2_v, c3_v, c4_v, c5_v)
        wid = lax.axis_index("s") * _NC + lax.axis_index("c")

        lane = lax.iota(jnp.int32, _L)

        for side, src in ((0, pos_hbm), (1, neg_hbm)):
            pltpu.sync_copy(src.at[pl.ds(wid * _PER_W, _PER_W)], pix_v)

            @pl.loop(0, _PER_W // _L)
            def _(m, side=side):
                rows = lane + m * _L
                sl = pl.ds(m * _L, _L)
                for col in range(3):
                    cv = jnp.full((_L,), col, jnp.int32)
                    col_bufs[3 * side + col][sl] = plsc.load_gather(
                        pix_v, [rows, cv])

        hacc_v[:] = jnp.zeros((_L,), jnp.float32)

        def fire(g, buf):
            sl = pl.ds(g * _C, _C)
            return tuple(
                pltpu.async_copy(table_hbm.at[col_bufs[s].at[sl]],
                                 row_bufs[s].at[buf], gsem[buf])
                for s in range(6))

        def wait(buf):
            for s in range(6):
                pltpu.make_async_copy(
                    table_hbm.at[col_bufs[s].at[pl.ds(0, _C)]],
                    row_bufs[s].at[buf], gsem[buf]).wait()

        def compute(buf):
            hp, tp, rp = hp_v.at[buf], tp_v.at[buf], rp_v.at[buf]
            hn, tn, rn = hn_v.at[buf], tn_v.at[buf], rn_v.at[buf]

            @pl.loop(0, _C, step=2)
            def _(i):
                for ii in range(2):
                    row = i + ii
                    accp = jnp.zeros((_L,), jnp.float32)
                    accn = jnp.zeros((_L,), jnp.float32)
                    for kk in range(_D // _L):
                        ds = pl.ds(kk * _L, _L)
                        dp = hp[row, ds] + rp[row, ds] - tp[row, ds]
                        dn = hn[row, ds] + rn[row, ds] - tn[row, ds]
                        accp = accp + dp * dp
                        accn = accn + dn * dn
                    pp_v[row, :] = accp
                    nn_v[row, :] = accn

            @pl.loop(0, _C // _L)
            def _(m):
                rows = lane + m * _L
                p2 = plsc.load_gather(pp_v, [rows, jnp.zeros((_L,), jnp.int32)])
                n2 = plsc.load_gather(nn_v, [rows, jnp.zeros((_L,), jnp.int32)])
                for l in range(1, _L):
                    cl = jnp.full((_L,), l, jnp.int32)
                    p2 = p2 + plsc.load_gather(pp_v, [rows, cl])
                    n2 = n2 + plsc.load_gather(nn_v, [rows, cl])
                hinge = jnp.maximum(_MARGIN + _nr_sqrt(p2) - _nr_sqrt(n2), 0.0)
                hacc_v[:] = hacc_v[:] + hinge

        fire(0, 0)

        @pl.loop(0, _NCHUNK // 2)
        def _(p):
            g0 = 2 * p
            fire(g0 + 1, 1)
            wait(0)
            compute(0)

            @pl.when(p < _NCHUNK // 2 - 1)
            def _():
                fire(g0 + 2, 0)

            wait(1)
            compute(1)

        pltpu.sync_copy(hacc_v, out_hbm.at[pl.ds(wid * _L, _L)])

    return k(table, pos_idx, neg_idx)


def _tc_sum(parts):
    def body(p_ref, o_ref):
        o_ref[...] = jnp.sum(p_ref[...])[None, None]

    return pl.pallas_call(
        body,
        out_shape=jax.ShapeDtypeStruct((1, 1), jnp.float32),
    )(parts)


def kernel(positive_triples, negative_triples, embeddings):
    parts = _sc_hinge_partials(
        embeddings, positive_triples, negative_triples)
    loss = _tc_sum(parts.reshape(_NW, _L))
    return loss[0, 0]


# speedup vs baseline: 1.5634x; 1.5634x over previous
import dataclasses
import functools

import jax
import jax.numpy as jnp
from jax import lax
from jax.experimental import pallas as pl
from jax.experimental.pallas import tpu as pltpu
from jax.experimental.pallas import tpu_sc as plsc

_B = 16384
_D = 128
_L = 16
_MARGIN = 1.0
_NC, _NS = 2, 16
_NW = _NC * _NS
_PER_W = _B // _NW
_C = 32
_NCHUNK = _PER_W // _C


def _nr_sqrt(u):
    bits = lax.bitcast_convert_type(u, jnp.int32)
    seed = jnp.int32(0x5F3759DF) - lax.shift_right_logical(bits, 1)
    y = lax.bitcast_convert_type(seed, jnp.float32)
    half_u = 0.5 * u
    for _ in range(3):
        y = y * (1.5 - half_u * y * y)
    return u * y


def _sc_hinge_partials(table, idx6):
    mesh = plsc.VectorSubcoreMesh(core_axis_name="c", subcore_axis_name="s")
    cp = pltpu.CompilerParams()
    if "needs_layout_passes" in pltpu.CompilerParams.__dataclass_fields__:
        cp = dataclasses.replace(cp, needs_layout_passes=False)

    @functools.partial(
        pl.kernel,
        compiler_params=cp,
        out_type=jax.ShapeDtypeStruct((_NW * _L,), jnp.float32),
        mesh=mesh,
        scratch_types=[
            pltpu.VMEM((_PER_W,), jnp.int32),
            pltpu.VMEM((_PER_W,), jnp.int32),
            pltpu.VMEM((_PER_W,), jnp.int32),
            pltpu.VMEM((_PER_W,), jnp.int32),
            pltpu.VMEM((_PER_W,), jnp.int32),
            pltpu.VMEM((_PER_W,), jnp.int32),
            pltpu.VMEM((2, _C, _D), jnp.float32),
            pltpu.VMEM((2, _C, _D), jnp.float32),
            pltpu.VMEM((2, _C, _D), jnp.float32),
            pltpu.VMEM((2, _C, _D), jnp.float32),
            pltpu.VMEM((2, _C, _D), jnp.float32),
            pltpu.VMEM((2, _C, _D), jnp.float32),
            pltpu.VMEM((_C, _L), jnp.float32),
            pltpu.VMEM((_C, _L), jnp.float32),
            pltpu.VMEM((_L,), jnp.float32),
            pltpu.SemaphoreType.DMA,
            pltpu.SemaphoreType.DMA,
        ],
    )
    def k(table_hbm, idx_hbm, out_hbm,
          c0_v, c1_v, c2_v, c3_v, c4_v, c5_v,
          hp_v, tp_v, rp_v, hn_v, tn_v, rn_v,
          pp_v, nn_v, hacc_v, g0, g1):
        gsem = (g0, g1)
        row_bufs = (hp_v, tp_v, rp_v, hn_v, tn_v, rn_v)
        col_bufs = (c0_v, c1_v, c2_v, c3_v, c4_v, c5_v)
        wid = lax.axis_index("s") * _NC + lax.axis_index("c")

        lane = lax.iota(jnp.int32, _L)
        for s in range(6):
            pltpu.sync_copy(idx_hbm.at[pl.ds(s * _B + wid * _PER_W, _PER_W)],
                            col_bufs[s])

        hacc_v[:] = jnp.zeros((_L,), jnp.float32)

        def fire(g, buf):
            sl = pl.ds(g * _C, _C)
            return tuple(
                pltpu.async_copy(table_hbm.at[col_bufs[s].at[sl]],
                                 row_bufs[s].at[buf], gsem[buf])
                for s in range(6))

        def wait(buf):
            for s in range(6):
                pltpu.make_async_copy(
                    table_hbm.at[col_bufs[s].at[pl.ds(0, _C)]],
                    row_bufs[s].at[buf], gsem[buf]).wait()

        def compute(buf):
            hp, tp, rp = hp_v.at[buf], tp_v.at[buf], rp_v.at[buf]
            hn, tn, rn = hn_v.at[buf], tn_v.at[buf], rn_v.at[buf]

            @pl.loop(0, _C, step=2)
            def _(i):
                for ii in range(2):
                    row = i + ii
                    accp = jnp.zeros((_L,), jnp.float32)
                    accn = jnp.zeros((_L,), jnp.float32)
                    for kk in range(_D // _L):
                        ds = pl.ds(kk * _L, _L)
                        dp = hp[row, ds] + rp[row, ds] - tp[row, ds]
                        dn = hn[row, ds] + rn[row, ds] - tn[row, ds]
                        accp = accp + dp * dp
                        accn = accn + dn * dn
                    pp_v[row, :] = accp
                    nn_v[row, :] = accn

            @pl.loop(0, _C // _L)
            def _(m):
                rows = lane + m * _L
                p2 = plsc.load_gather(pp_v, [rows, jnp.zeros((_L,), jnp.int32)])
                n2 = plsc.load_gather(nn_v, [rows, jnp.zeros((_L,), jnp.int32)])
                for l in range(1, _L):
                    cl = jnp.full((_L,), l, jnp.int32)
                    p2 = p2 + plsc.load_gather(pp_v, [rows, cl])
                    n2 = n2 + plsc.load_gather(nn_v, [rows, cl])
                hinge = jnp.maximum(_MARGIN + _nr_sqrt(p2) - _nr_sqrt(n2), 0.0)
                hacc_v[:] = hacc_v[:] + hinge

        fire(0, 0)

        @pl.loop(0, _NCHUNK // 2)
        def _(p):
            g0 = 2 * p
            fire(g0 + 1, 1)
            wait(0)
            compute(0)

            @pl.when(p < _NCHUNK // 2 - 1)
            def _():
                fire(g0 + 2, 0)

            wait(1)
            compute(1)

        pltpu.sync_copy(hacc_v, out_hbm.at[pl.ds(wid * _L, _L)])

    return k(table, idx6)


def _tc_sum(parts):
    def body(p_ref, o_ref):
        o_ref[...] = jnp.sum(p_ref[...])[None, None]

    return pl.pallas_call(
        body,
        out_shape=jax.ShapeDtypeStruct((1, 1), jnp.float32),
    )(parts)


def kernel(positive_triples, negative_triples, embeddings):
    idx6 = jnp.concatenate([
        positive_triples[:, 0], positive_triples[:, 1], positive_triples[:, 2],
        negative_triples[:, 0], negative_triples[:, 1], negative_triples[:, 2],
    ]).astype(jnp.int32)
    parts = _sc_hinge_partials(embeddings, idx6)
    loss = _tc_sum(parts.reshape(_NW, _L))
    return loss[0, 0]


# speedup vs baseline: 1.6005x; 1.0237x over previous
import dataclasses
import functools

import jax
import jax.numpy as jnp
from jax import lax
from jax.experimental import pallas as pl
from jax.experimental.pallas import tpu as pltpu
from jax.experimental.pallas import tpu_sc as plsc

_B = 16384
_D = 128
_L = 16
_MARGIN = 1.0
_NC, _NS = 2, 16
_NW = _NC * _NS
_PER_W = _B // _NW
_C = 32
_NCHUNK = _PER_W // _C


def _nr_sqrt(u):
    bits = lax.bitcast_convert_type(u, jnp.int32)
    seed = jnp.int32(0x5F3759DF) - lax.shift_right_logical(bits, 1)
    y = lax.bitcast_convert_type(seed, jnp.float32)
    half_u = 0.5 * u
    for _ in range(3):
        y = y * (1.5 - half_u * y * y)
    return u * y


def _sc_hinge_partials(table, idx6):
    mesh = plsc.VectorSubcoreMesh(core_axis_name="c", subcore_axis_name="s")
    cp = pltpu.CompilerParams()
    if "needs_layout_passes" in pltpu.CompilerParams.__dataclass_fields__:
        cp = dataclasses.replace(cp, needs_layout_passes=False)

    @functools.partial(
        pl.kernel,
        compiler_params=cp,
        out_type=jax.ShapeDtypeStruct((_NW * _L,), jnp.float32),
        mesh=mesh,
        scratch_types=[
            pltpu.VMEM((_PER_W,), jnp.int32),
            pltpu.VMEM((_PER_W,), jnp.int32),
            pltpu.VMEM((_PER_W,), jnp.int32),
            pltpu.VMEM((_PER_W,), jnp.int32),
            pltpu.VMEM((_PER_W,), jnp.int32),
            pltpu.VMEM((_PER_W,), jnp.int32),
            pltpu.VMEM((2, _C, _D), jnp.float32),
            pltpu.VMEM((2, _C, _D), jnp.float32),
            pltpu.VMEM((2, _C, _D), jnp.float32),
            pltpu.VMEM((2, _C, _D), jnp.float32),
            pltpu.VMEM((2, _C, _D), jnp.float32),
            pltpu.VMEM((2, _C, _D), jnp.float32),
            pltpu.VMEM((_C, _L), jnp.float32),
            pltpu.VMEM((_C, _L), jnp.float32),
            pltpu.VMEM((_L,), jnp.float32),
            pltpu.SemaphoreType.DMA,
            pltpu.SemaphoreType.DMA,
        ],
    )
    def k(table_hbm, idx_hbm, out_hbm,
          c0_v, c1_v, c2_v, c3_v, c4_v, c5_v,
          hp_v, tp_v, rp_v, hn_v, tn_v, rn_v,
          pp_v, nn_v, hacc_v, g0, g1):
        gsem = (g0, g1)
        row_bufs = (hp_v, tp_v, rp_v, hn_v, tn_v, rn_v)
        col_bufs = (c0_v, c1_v, c2_v, c3_v, c4_v, c5_v)
        wid = lax.axis_index("s") * _NC + lax.axis_index("c")

        lane = lax.iota(jnp.int32, _L)
        for s in range(6):
            pltpu.sync_copy(idx_hbm.at[pl.ds(s * _B + wid * _PER_W, _PER_W)],
                            col_bufs[s])

        hacc_v[:] = jnp.zeros((_L,), jnp.float32)

        def fire(g, buf):
            sl = pl.ds(g * _C, _C)
            return tuple(
                pltpu.async_copy(table_hbm.at[col_bufs[s].at[sl]],
                                 row_bufs[s].at[buf], gsem[buf])
                for s in range(6))

        def wait(buf):
            for s in range(6):
                pltpu.make_async_copy(
                    table_hbm.at[col_bufs[s].at[pl.ds(0, _C)]],
                    row_bufs[s].at[buf], gsem[buf]).wait()

        def compute(buf):
            hp, tp, rp = hp_v.at[buf], tp_v.at[buf], rp_v.at[buf]
            hn, tn, rn = hn_v.at[buf], tn_v.at[buf], rn_v.at[buf]

            @pl.loop(0, _C, step=2)
            def _(i):
                for ii in range(2):
                    row = i + ii
                    accp = jnp.zeros((_L,), jnp.float32)
                    accn = jnp.zeros((_L,), jnp.float32)
                    for kk in range(_D // _L):
                        ds = pl.ds(kk * _L, _L)
                        dp = hp[row, ds] + rp[row, ds] - tp[row, ds]
                        dn = hn[row, ds] + rn[row, ds] - tn[row, ds]
                        accp = accp + dp * dp
                        accn = accn + dn * dn
                    pp_v[row, :] = accp
                    nn_v[row, :] = accn

            @pl.loop(0, _C // _L)
            def _(m):
                rows = lane + m * _L
                p2 = plsc.load_gather(pp_v, [rows, jnp.zeros((_L,), jnp.int32)])
                n2 = plsc.load_gather(nn_v, [rows, jnp.zeros((_L,), jnp.int32)])
                for l in range(1, _L):
                    cl = jnp.full((_L,), l, jnp.int32)
                    p2 = p2 + plsc.load_gather(pp_v, [rows, cl])
                    n2 = n2 + plsc.load_gather(nn_v, [rows, cl])
                hinge = jnp.maximum(_MARGIN + _nr_sqrt(p2) - _nr_sqrt(n2), 0.0)
                hacc_v[:] = hacc_v[:] + hinge

        fire(0, 0)

        @pl.loop(0, _NCHUNK // 2)
        def _(p):
            g0 = 2 * p
            fire(g0 + 1, 1)
            wait(0)
            compute(0)

            @pl.when(p < _NCHUNK // 2 - 1)
            def _():
                fire(g0 + 2, 0)

            wait(1)
            compute(1)

        pltpu.sync_copy(hacc_v, out_hbm.at[pl.ds(wid * _L, _L)])

    return k(table, idx6)


def _tc_sum(parts):
    def body(p_ref, o_ref):
        o_ref[...] = jnp.sum(p_ref[...])[None, None]

    return pl.pallas_call(
        body,
        out_shape=jax.ShapeDtypeStruct((1, 1), jnp.float32),
    )(parts)


def kernel(positive_triples, negative_triples, embeddings):
    idx6 = jnp.concatenate([
        positive_triples[:, 0], positive_triples[:, 1], positive_triples[:, 2],
        negative_triples[:, 0], negative_triples[:, 1], negative_triples[:, 2],
    ]).astype(jnp.int32)
    parts = _sc_hinge_partials(embeddings, idx6)
    loss = _tc_sum(parts)
    return loss[0, 0]
